# baseline (speedup 1.0000x reference)
import functools

import jax
import jax.numpy as jnp
from jax import lax
from jax.experimental import pallas as pl
from jax.experimental.pallas import tpu as pltpu
from jax.experimental.pallas import tpu_sc as plsc

_ROWS = 16384
_OUT_SIZE = 1024
_IN_COLS = 128
_ZCOLS = _OUT_SIZE - _IN_COLS
_NW = 32
_RPW = _ROWS // _NW
_CHUNK = 128
_NCHUNK = _RPW // _CHUNK
_ZROWS = 64
_NZ = _RPW // _ZROWS


def _sc_body(x_hbm, w_hbm, out_hbm, xa, xb, oa, ob, z_v, w_v,
             rsa, rsb, wsa, wsb, zsem):
    wid = lax.axis_index("s") * 2 + lax.axis_index("c")
    base = wid * _RPW

    def x_src(c):
        return x_hbm.at[pl.ds(base + c * _CHUNK, _CHUNK), pl.ds(0, _IN_COLS)]

    xbufs, obufs = [xa, xb], [oa, ob]
    rsems, wsems = [rsa, rsb], [wsa, wsb]
    reads = [None, None]
    writes = [None, None]

    reads[0] = pltpu.async_copy(x_src(0), xbufs[0], rsems[0])
    reads[1] = pltpu.async_copy(x_src(1), xbufs[1], rsems[1])

    pltpu.sync_copy(w_hbm, w_v)

    zeros16 = jnp.zeros((16,), jnp.float32)

    def zrow(r, _):
        for j in range(_ZCOLS // 16):
            z_v[r, pl.ds(j * 16, 16)] = zeros16
        return 0

    lax.fori_loop(0, _ZROWS, zrow, 0)

    zwrites = [
        pltpu.async_copy(
            z_v,
            out_hbm.at[pl.ds(base + c * _ZROWS, _ZROWS), pl.ds(_IN_COLS, _ZCOLS)],
            zsem)
        for c in range(_NZ)
    ]

    def orow(r, _):
        for j in range(4):
            oa[r, pl.ds(64 + j * 16, 16)] = zeros16
            ob[r, pl.ds(64 + j * 16, 16)] = zeros16
        return 0

    lax.fori_loop(0, _CHUNK, orow, 0)

    w0 = [w_v[pl.ds(j * 16, 16)] for j in range(4)]
    w1 = [w_v[pl.ds(64 + j * 16, 16)] for j in range(4)]

    def compute(x_v, o_v):
        def row_body(r, _):
            for j in range(4):
                a = x_v[r, pl.ds(j * 16, 16)] * w0[j]
                b = x_v[r, pl.ds(64 + j * 16, 16)] * w1[j]
                o_v[r, pl.ds(j * 16, 16)] = a + b
            return 0
        lax.fori_loop(0, _CHUNK, row_body, 0)

    for c in range(_NCHUNK):
        b = c & 1
        reads[b].wait()
        if writes[b] is not None:
            writes[b].wait()
        compute(xbufs[b], obufs[b])
        writes[b] = pltpu.async_copy(
            obufs[b],
            out_hbm.at[pl.ds(base + c * _CHUNK, _CHUNK), pl.ds(0, _IN_COLS)],
            wsems[b])
        if c + 2 < _NCHUNK:
            reads[b] = pltpu.async_copy(x_src(c + 2), xbufs[b], rsems[b])
    writes[0].wait()
    writes[1].wait()
    for zw in zwrites:
        zw.wait()


def kernel(x, weights):
    w_flat = weights.reshape(_IN_COLS)
    mesh = plsc.VectorSubcoreMesh(core_axis_name="c", subcore_axis_name="s")
    run = functools.partial(
        pl.kernel,
        mesh=mesh,
        out_type=jax.ShapeDtypeStruct((_ROWS, _OUT_SIZE), jnp.float32),
        scratch_types=[
            pltpu.VMEM((_CHUNK, _IN_COLS), jnp.float32),
            pltpu.VMEM((_CHUNK, _IN_COLS), jnp.float32),
            pltpu.VMEM((_CHUNK, _IN_COLS), jnp.float32),
            pltpu.VMEM((_CHUNK, _IN_COLS), jnp.float32),
            pltpu.VMEM((_ZROWS, _ZCOLS), jnp.float32),
            pltpu.VMEM((_IN_COLS,), jnp.float32),
            pltpu.SemaphoreType.DMA,
            pltpu.SemaphoreType.DMA,
            pltpu.SemaphoreType.DMA,
            pltpu.SemaphoreType.DMA,
            pltpu.SemaphoreType.DMA,
        ],
    )(_sc_body)
    return run(x, w_flat)


# speedup vs baseline: 1.0208x; 1.0208x over previous
import functools

import jax
import jax.numpy as jnp
from jax import lax
from jax.experimental import pallas as pl
from jax.experimental.pallas import tpu as pltpu
from jax.experimental.pallas import tpu_sc as plsc

_ROWS = 16384
_OUT_SIZE = 1024
_IN_COLS = 128
_ZCOLS = _OUT_SIZE - _IN_COLS
_NW = 32
_RPW = _ROWS // _NW
_CHUNK = 128
_NCHUNK = _RPW // _CHUNK
_ZROWS = 32
_NZ = _RPW // _ZROWS


def _sc_body(x_hbm, w_hbm, out_hbm, xa, xb, oa, ob, z_v, w_v,
             rsa, rsb, wsa, wsb, zsem):
    wid = lax.axis_index("c") * 16 + lax.axis_index("s")
    base = wid * _RPW

    def x_src(c):
        return x_hbm.at[pl.ds(base + c * _CHUNK, _CHUNK), pl.ds(0, _IN_COLS)]

    xbufs, obufs = [xa, xb], [oa, ob]
    rsems, wsems = [rsa, rsb], [wsa, wsb]
    reads = [None, None]
    writes = [None, None]

    reads[0] = pltpu.async_copy(x_src(0), xbufs[0], rsems[0])
    reads[1] = pltpu.async_copy(x_src(1), xbufs[1], rsems[1])

    pltpu.sync_copy(w_hbm, w_v)

    zeros16 = jnp.zeros((16,), jnp.float32)

    def zrow(r, _):
        for j in range(_ZCOLS // 16):
            z_v[r, pl.ds(j * 16, 16)] = zeros16
        return 0

    lax.fori_loop(0, _ZROWS, zrow, 0)

    zwrites = [
        pltpu.async_copy(
            z_v,
            out_hbm.at[pl.ds(base + c * _ZROWS, _ZROWS), pl.ds(_IN_COLS, _ZCOLS)],
            zsem)
        for c in range(_NZ)
    ]

    def orow(r, _):
        for j in range(4):
            oa[r, pl.ds(64 + j * 16, 16)] = zeros16
            ob[r, pl.ds(64 + j * 16, 16)] = zeros16
        return 0

    lax.fori_loop(0, _CHUNK, orow, 0)

    w0 = [w_v[pl.ds(j * 16, 16)] for j in range(4)]
    w1 = [w_v[pl.ds(64 + j * 16, 16)] for j in range(4)]

    def compute(x_v, o_v):
        def row_body(r, _):
            for j in range(4):
                a = x_v[r, pl.ds(j * 16, 16)] * w0[j]
                b = x_v[r, pl.ds(64 + j * 16, 16)] * w1[j]
                o_v[r, pl.ds(j * 16, 16)] = a + b
            return 0
        lax.fori_loop(0, _CHUNK, row_body, 0)

    for c in range(_NCHUNK):
        b = c & 1
        reads[b].wait()
        if writes[b] is not None:
            writes[b].wait()
        compute(xbufs[b], obufs[b])
        writes[b] = pltpu.async_copy(
            obufs[b],
            out_hbm.at[pl.ds(base + c * _CHUNK, _CHUNK), pl.ds(0, _IN_COLS)],
            wsems[b])
        if c + 2 < _NCHUNK:
            reads[b] = pltpu.async_copy(x_src(c + 2), xbufs[b], rsems[b])
    writes[0].wait()
    writes[1].wait()
    for zw in zwrites:
        zw.wait()


def kernel(x, weights):
    w_flat = weights.reshape(_IN_COLS)
    mesh = plsc.VectorSubcoreMesh(core_axis_name="c", subcore_axis_name="s")
    run = functools.partial(
        pl.kernel,
        mesh=mesh,
        out_type=jax.ShapeDtypeStruct((_ROWS, _OUT_SIZE), jnp.float32),
        scratch_types=[
            pltpu.VMEM((_CHUNK, _IN_COLS), jnp.float32),
            pltpu.VMEM((_CHUNK, _IN_COLS), jnp.float32),
            pltpu.VMEM((_CHUNK, _IN_COLS), jnp.float32),
            pltpu.VMEM((_CHUNK, _IN_COLS), jnp.float32),
            pltpu.VMEM((_ZROWS, _ZCOLS), jnp.float32),
            pltpu.VMEM((_IN_COLS,), jnp.float32),
            pltpu.SemaphoreType.DMA,
            pltpu.SemaphoreType.DMA,
            pltpu.SemaphoreType.DMA,
            pltpu.SemaphoreType.DMA,
            pltpu.SemaphoreType.DMA,
        ],
    )(_sc_body)
    return run(x, w_flat)
